# Initial kernel scaffold; baseline (speedup 1.0000x reference)
import functools

import jax
import jax.numpy as jnp
from jax import lax
from jax.experimental import pallas as pl
from jax.experimental.pallas import tpu as pltpu
from jax.experimental.pallas import tpu_sc as plsc

NC, NS, L = 2, 16, 16
NW = NC * NS


@functools.partial(jax.jit, static_argnames=("M", "N", "D"))
def _sc_logits(emb, plab, nlab, W, *, M, N, D):
    ntok = M // NW
    CT = 16
    nchunks = ntok // CT
    mesh = plsc.VectorSubcoreMesh(
        core_axis_name="c", subcore_axis_name="s", num_cores=NC, num_subcores=NS
    )

    @functools.partial(
        pl.kernel,
        out_type=[
            jax.ShapeDtypeStruct((M,), jnp.float32),
            jax.ShapeDtypeStruct((M, N), jnp.float32),
        ],
        mesh=mesh,
        scratch_types=[
            pltpu.VMEM((CT, N), jnp.int32),
            pltpu.VMEM((CT,), jnp.int32),
            pltpu.VMEM((CT, D), jnp.float32),
            pltpu.VMEM((CT, D), jnp.float32),
            pltpu.VMEM((N, D), jnp.float32),
            pltpu.VMEM((CT,), jnp.float32),
            pltpu.VMEM((CT, N), jnp.float32),
            pltpu.SemaphoreType.DMA,
        ],
    )
    def body(emb_hbm, plab_hbm, nlab_hbm, w_hbm, pos_out, neg_out,
             lab_v, plab_v, embc_v, prows_v, rows_v, pout_v, nout_v, sem):
        wid = lax.axis_index("s") * NC + lax.axis_index("c")
        base = wid * ntok

        def chunk_body(ci, _):
            row0 = base + ci * CT
            pltpu.sync_copy(nlab_hbm.at[pl.ds(row0, CT)], lab_v)
            pltpu.sync_copy(plab_hbm.at[pl.ds(row0, CT)], plab_v)
            pltpu.sync_copy(emb_hbm.at[pl.ds(row0, CT)], embc_v)
            pltpu.async_copy(w_hbm.at[plab_v], prows_v, sem).wait()

            def tok_body(t, _):
                pltpu.async_copy(w_hbm.at[lab_v.at[t]], rows_v, sem).wait()
                e0 = embc_v[t, pl.ds(0, L)]
                e1 = embc_v[t, pl.ds(L, L)]
                e2 = embc_v[t, pl.ds(2 * L, L)]
                e3 = embc_v[t, pl.ds(3 * L, L)]
                pacc = (prows_v[t, pl.ds(0, L)] * e0
                        + prows_v[t, pl.ds(L, L)] * e1
                        + prows_v[t, pl.ds(2 * L, L)] * e2
                        + prows_v[t, pl.ds(3 * L, L)] * e3)
                pout_v[t] = jnp.sum(pacc)

                def k_body(k, _):
                    acc = (rows_v[k, pl.ds(0, L)] * e0
                           + rows_v[k, pl.ds(L, L)] * e1
                           + rows_v[k, pl.ds(2 * L, L)] * e2
                           + rows_v[k, pl.ds(3 * L, L)] * e3)
                    nout_v[t, k] = jnp.sum(acc)
                    return _

                return lax.fori_loop(0, N, k_body, _)

            lax.fori_loop(0, CT, tok_body, None)
            pltpu.sync_copy(pout_v, pos_out.at[pl.ds(row0, CT)])
            pltpu.sync_copy(nout_v, neg_out.at[pl.ds(row0, CT)])
            return _

        lax.fori_loop(0, nchunks, chunk_body, None)

    return body(emb, plab, nlab, W)


def kernel(model_embeddings, positive_labels, negative_labels,
           target_padding_mask, W):
    B, S, D = model_embeddings.shape
    N = negative_labels.shape[-1]
    M = B * S
    emb = model_embeddings.reshape(M, D)
    plab = positive_labels.reshape(M).astype(jnp.int32)
    nlab = negative_labels.reshape(M, N).astype(jnp.int32)
    pos_logits, neg_logits = _sc_logits(emb, plab, nlab, W, M=M, N=N, D=D)
    return (pos_logits.reshape(M, 1), neg_logits,
            positive_labels.reshape(M, 1), negative_labels.reshape(M, N))


# speedup vs baseline: 32.4803x; 32.4803x over previous
import functools

import jax
import jax.numpy as jnp
from jax import lax
from jax.experimental import pallas as pl
from jax.experimental.pallas import tpu as pltpu
from jax.experimental.pallas import tpu_sc as plsc

NC, NS, L = 2, 16, 16
NW = NC * NS


@functools.partial(jax.jit, static_argnames=("M", "N", "D"))
def _sc_logits(emb_f, plab, nlab_f, W, *, M, N, D):
    ntok = M // NW
    CT = L
    nchunks = ntok // CT
    mesh = plsc.VectorSubcoreMesh(
        core_axis_name="c", subcore_axis_name="s", num_cores=NC, num_subcores=NS
    )

    @functools.partial(
        pl.kernel,
        out_type=[
            jax.ShapeDtypeStruct((M,), jnp.float32),
            jax.ShapeDtypeStruct((M * N,), jnp.float32),
        ],
        mesh=mesh,
        compiler_params=pltpu.CompilerParams(
            needs_layout_passes=False, use_tc_tiling_on_sc=False),
        scratch_types=[
            pltpu.VMEM((CT * N,), jnp.int32),
            pltpu.VMEM((CT,), jnp.int32),
            pltpu.VMEM((CT * D,), jnp.float32),
            pltpu.VMEM((CT, D), jnp.float32),
            pltpu.VMEM((N, D), jnp.float32),
            pltpu.VMEM((CT,), jnp.float32),
            pltpu.VMEM((CT * N,), jnp.float32),
            pltpu.SemaphoreType.DMA,
        ],
    )
    def body(emb_hbm, plab_hbm, nlab_hbm, w_hbm, pos_out, neg_out,
             lab_v, plab_v, embc_v, prows_v, rows_v, pout_v, nout_v, sem):
        wid = lax.axis_index("s") * NC + lax.axis_index("c")
        base = wid * ntok
        lane = lax.iota(jnp.int32, L)

        def chunk_body(ci, _):
            row0 = base + ci * CT
            pltpu.sync_copy(nlab_hbm.at[pl.ds(row0 * N, CT * N)], lab_v)
            pltpu.sync_copy(plab_hbm.at[pl.ds(row0, CT)], plab_v)
            pltpu.sync_copy(emb_hbm.at[pl.ds(row0 * D, CT * D)], embc_v)
            pltpu.async_copy(w_hbm.at[plab_v], prows_v, sem).wait()

            def tok_body(t, pres):
                pltpu.async_copy(w_hbm.at[lab_v.at[pl.ds(t * N, N)]],
                                 rows_v, sem).wait()
                ev = [embc_v[pl.ds(t * D + j * L, L)] for j in range(D // L)]

                pacc = (prows_v[t, pl.ds(0, L)] * ev[0]
                        + prows_v[t, pl.ds(L, L)] * ev[1]
                        + prows_v[t, pl.ds(2 * L, L)] * ev[2]
                        + prows_v[t, pl.ds(3 * L, L)] * ev[3])
                pres = jnp.where(lane == t, jnp.sum(pacc), pres)

                def grp_body(g, _):
                    res = jnp.zeros((L,), jnp.float32)
                    for j in range(L):
                        k = g * L + j
                        acc = (rows_v[k, pl.ds(0, L)] * ev[0]
                               + rows_v[k, pl.ds(L, L)] * ev[1]
                               + rows_v[k, pl.ds(2 * L, L)] * ev[2]
                               + rows_v[k, pl.ds(3 * L, L)] * ev[3])
                        res = jnp.where(lane == j, jnp.sum(acc), res)
                    nout_v[pl.ds(t * N + g * L, L)] = res
                    return _

                lax.fori_loop(0, N // L, grp_body, 0)
                return pres

            pres = lax.fori_loop(0, CT, tok_body, jnp.zeros((L,), jnp.float32))
            pout_v[...] = pres
            pltpu.sync_copy(pout_v, pos_out.at[pl.ds(row0, CT)])
            pltpu.sync_copy(nout_v, neg_out.at[pl.ds(row0 * N, CT * N)])
            return _

        lax.fori_loop(0, nchunks, chunk_body, 0)

    return body(emb_f, plab, nlab_f, W)


def kernel(model_embeddings, positive_labels, negative_labels,
           target_padding_mask, W):
    B, S, D = model_embeddings.shape
    N = negative_labels.shape[-1]
    M = B * S
    emb_f = model_embeddings.reshape(M * D)
    plab = positive_labels.reshape(M).astype(jnp.int32)
    nlab_f = negative_labels.reshape(M * N).astype(jnp.int32)
    pos_logits, neg_logits = _sc_logits(emb_f, plab, nlab_f, W,
                                        M=M, N=N, D=D)
    return (pos_logits.reshape(M, 1), neg_logits.reshape(M, N),
            positive_labels.reshape(M, 1), negative_labels.reshape(M, N))


# speedup vs baseline: 47.4596x; 1.4612x over previous
import functools

import jax
import jax.numpy as jnp
from jax import lax
from jax.experimental import pallas as pl
from jax.experimental.pallas import tpu as pltpu
from jax.experimental.pallas import tpu_sc as plsc

NC, NS, L = 2, 16, 16
NW = NC * NS


@functools.partial(jax.jit, static_argnames=("M", "N", "D"))
def _sc_logits(emb_f, plab, nlab_f, W, *, M, N, D):
    ntok = M // NW
    CT = L
    nchunks = ntok // CT
    mesh = plsc.VectorSubcoreMesh(
        core_axis_name="c", subcore_axis_name="s", num_cores=NC, num_subcores=NS
    )

    @functools.partial(
        pl.kernel,
        out_type=[
            jax.ShapeDtypeStruct((M,), jnp.float32),
            jax.ShapeDtypeStruct((M * N,), jnp.float32),
        ],
        mesh=mesh,
        compiler_params=pltpu.CompilerParams(
            needs_layout_passes=False, use_tc_tiling_on_sc=False),
        scratch_types=[
            pltpu.VMEM((CT * N,), jnp.int32),
            pltpu.VMEM((CT,), jnp.int32),
            pltpu.VMEM((CT * D,), jnp.float32),
            pltpu.VMEM((CT, D), jnp.float32),
            pltpu.VMEM((N, D), jnp.float32),
            pltpu.VMEM((N, D), jnp.float32),
            pltpu.VMEM((CT,), jnp.float32),
            pltpu.VMEM((CT * N,), jnp.float32),
            pltpu.SemaphoreType.DMA,
            pltpu.SemaphoreType.DMA,
            pltpu.SemaphoreType.DMA,
        ],
    )
    def body(emb_hbm, plab_hbm, nlab_hbm, w_hbm, pos_out, neg_out,
             lab_v, plab_v, embc_v, prows_v, rows_a, rows_b, pout_v, nout_v,
             sem, sem_a, sem_b):
        wid = lax.axis_index("s") * NC + lax.axis_index("c")
        base = wid * ntok
        lane = lax.iota(jnp.int32, L)

        def chunk_body(ci, _):
            row0 = base + ci * CT
            pltpu.sync_copy(nlab_hbm.at[pl.ds(row0 * N, CT * N)], lab_v)
            pltpu.sync_copy(plab_hbm.at[pl.ds(row0, CT)], plab_v)
            pltpu.sync_copy(emb_hbm.at[pl.ds(row0 * D, CT * D)], embc_v)
            pltpu.async_copy(w_hbm.at[plab_v], prows_v, sem).wait()

            def compute_tok(t, rows_v, pres):
                ev = [embc_v[pl.ds(t * D + j * L, L)] for j in range(D // L)]

                pacc = (prows_v[t, pl.ds(0, L)] * ev[0]
                        + prows_v[t, pl.ds(L, L)] * ev[1]
                        + prows_v[t, pl.ds(2 * L, L)] * ev[2]
                        + prows_v[t, pl.ds(3 * L, L)] * ev[3])
                pres = jnp.where(lane == t, jnp.sum(pacc), pres)

                def grp_body(g, _):
                    res = jnp.zeros((L,), jnp.float32)
                    for j in range(L):
                        k = g * L + j
                        acc = (rows_v[k, pl.ds(0, L)] * ev[0]
                               + rows_v[k, pl.ds(L, L)] * ev[1]
                               + rows_v[k, pl.ds(2 * L, L)] * ev[2]
                               + rows_v[k, pl.ds(3 * L, L)] * ev[3])
                        res = jnp.where(lane == j, jnp.sum(acc), res)
                    nout_v[pl.ds(t * N + g * L, L)] = res
                    return _

                lax.fori_loop(0, N // L, grp_body, 0)
                return pres

            def issue(t, rows_v, s):
                pltpu.async_copy(w_hbm.at[lab_v.at[pl.ds(t * N, N)]],
                                 rows_v, s)

            def wait(t, rows_v, s):
                pltpu.make_async_copy(w_hbm.at[lab_v.at[pl.ds(t * N, N)]],
                                      rows_v, s).wait()

            issue(0, rows_a, sem_a)

            def pair_body(p, pres):
                t0 = 2 * p
                issue(t0 + 1, rows_b, sem_b)
                wait(t0, rows_a, sem_a)
                pres = compute_tok(t0, rows_a, pres)

                @pl.when(p < CT // 2 - 1)
                def _():
                    issue(t0 + 2, rows_a, sem_a)

                wait(t0 + 1, rows_b, sem_b)
                return compute_tok(t0 + 1, rows_b, pres)

            pres = lax.fori_loop(0, CT // 2, pair_body,
                                 jnp.zeros((L,), jnp.float32))
            pout_v[...] = pres
            pltpu.sync_copy(pout_v, pos_out.at[pl.ds(row0, CT)])
            pltpu.sync_copy(nout_v, neg_out.at[pl.ds(row0 * N, CT * N)])
            return _

        lax.fori_loop(0, nchunks, chunk_body, 0)

    return body(emb_f, plab, nlab_f, W)


def kernel(model_embeddings, positive_labels, negative_labels,
           target_padding_mask, W):
    B, S, D = model_embeddings.shape
    N = negative_labels.shape[-1]
    M = B * S
    emb_f = model_embeddings.reshape(M * D)
    plab = positive_labels.reshape(M).astype(jnp.int32)
    nlab_f = negative_labels.reshape(M * N).astype(jnp.int32)
    pos_logits, neg_logits = _sc_logits(emb_f, plab, nlab_f, W,
                                        M=M, N=N, D=D)
    return (pos_logits.reshape(M, 1), neg_logits.reshape(M, N),
            positive_labels.reshape(M, 1), negative_labels.reshape(M, N))
